# Initial kernel scaffold; baseline (speedup 1.0000x reference)
import jax
import jax.numpy as jnp
from jax.experimental import pallas as pl


def kernel(inputs, idxs, targets, cams, centers, excenters):
    raise NotImplementedError("write your pallas kernel here")


# speedup vs baseline: 1.0369x; 1.0369x over previous
import functools

import jax
import jax.numpy as jnp
from jax.experimental import pallas as pl
from jax.experimental.pallas import tpu as pltpu


def _loss_kernel(gids_ref, x_ref, centers_ref, exc_ref, tgt_ref, out_ref,
                 s1_acc, s2_acc, *, n_steps, blk, k_per_group, n_groups,
                 inv_tau):
    i = pl.program_id(0)

    @pl.when(i == 0)
    def _init():
        s1_acc[:, :] = jnp.zeros_like(s1_acc)
        s2_acc[:, :] = jnp.zeros_like(s2_acc)

    x = x_ref[:, :]
    eb = jax.lax.dot_general(
        x, exc_ref[:, :],
        dimension_numbers=(((1,), (1,)), ((), ())),
        preferred_element_type=jnp.float32)
    ee = jnp.exp(eb * inv_tau)

    col = i * blk + jax.lax.broadcasted_iota(jnp.int32, ee.shape, 1)
    col_cluster = col // k_per_group
    member = col_cluster == gids_ref[0]
    for g in range(1, n_groups):
        member = member | (col_cluster == gids_ref[g])

    s2_acc[:, :] += jnp.sum(ee, axis=1, keepdims=True)
    s1_acc[:, :] += jnp.sum(jnp.where(member, ee, 0.0), axis=1, keepdims=True)

    @pl.when(i == n_steps - 1)
    def _finalize():
        b = x.shape[0]
        co = jax.lax.dot_general(
            x, centers_ref[:, :],
            dimension_numbers=(((1,), (1,)), ((), ())),
            preferred_element_type=jnp.float32)
        se = jnp.sum(jnp.exp(co * inv_tau), axis=1)
        tgt = tgt_ref[0, :]
        cols = jax.lax.broadcasted_iota(jnp.int32, co.shape, 1)
        onehot = cols == tgt[:, None]
        out_t = jnp.sum(jnp.where(onehot, co, 0.0), axis=1)
        nce = -jnp.mean(out_t * inv_tau - jnp.log(se))
        l2 = jnp.mean(jnp.log(s2_acc[:, 0]) - jnp.log(s1_acc[:, 0]))
        out_ref[0, 0] = nce + l2


def kernel(inputs, idxs, targets, cams, centers, excenters):
    del idxs, cams
    b, d = inputs.shape
    c = centers.shape[0]
    _, k, _ = excenters.shape
    n_groups = b // k
    ck = excenters.shape[0] * k

    blk = 2048
    n_steps = ck // blk

    exc2d = excenters.reshape(ck, d)
    gids = targets.reshape(n_groups, k)[:, 0]
    tgt2d = targets.reshape(1, b)

    grid_spec = pltpu.PrefetchScalarGridSpec(
        num_scalar_prefetch=1,
        grid=(n_steps,),
        in_specs=[
            pl.BlockSpec((b, d), lambda i, g: (0, 0)),
            pl.BlockSpec((c, d), lambda i, g: (0, 0)),
            pl.BlockSpec((blk, d), lambda i, g: (i, 0)),
            pl.BlockSpec((1, b), lambda i, g: (0, 0)),
        ],
        out_specs=pl.BlockSpec(memory_space=pltpu.SMEM),
        scratch_shapes=[
            pltpu.VMEM((b, 1), jnp.float32),
            pltpu.VMEM((b, 1), jnp.float32),
        ],
    )

    fn = functools.partial(
        _loss_kernel, n_steps=n_steps, blk=blk, k_per_group=k,
        n_groups=n_groups, inv_tau=20.0)

    out = pl.pallas_call(
        fn,
        grid_spec=grid_spec,
        out_shape=jax.ShapeDtypeStruct((1, 1), jnp.float32),
    )(gids, inputs, centers, exc2d, tgt2d)
    return out[0, 0]
